# baseline (speedup 1.0000x reference)
import functools

import jax
import jax.numpy as jnp
from jax import lax
from jax.experimental import pallas as pl
from jax.experimental.pallas import tpu as pltpu
from jax.experimental.pallas import tpu_sc as plsc

D_MODEL = 768
N_OUTCOMES = 5
OUTCOME_TOKEN_BASE = 4273
VOCAB = 4278

ROW_BLK = 1088
VOCAB_PAD = 4352

W_COLS = 144
SRC_OFF = 0
DST_OFF = 64
PROMO_OFF = 128
OUTCOME_OFF = 133
PAD_COL = 138

NUM_CORES = 2
NUM_SUBCORES = 16
NUM_WORKERS = NUM_CORES * NUM_SUBCORES
TOKENS = 4 * 8192
IDS_PER_WORKER = TOKENS // NUM_WORKERS
CHUNK = 64
NBUF = 2
NUM_CHUNKS = IDS_PER_WORKER // CHUNK


def _build_table_kernel(src_ref, dst_ref, promo_ref, outc_ref, pad_ref, out_ref, w_ref):
    i = pl.program_id(0)

    @pl.when(i == 0)
    def _concat_w():
        w_ref[SRC_OFF : SRC_OFF + 64, :] = src_ref[:, :]
        w_ref[DST_OFF : DST_OFF + 64, :] = dst_ref[:, :]
        w_ref[PROMO_OFF : PROMO_OFF + N_OUTCOMES, :] = promo_ref[:, :]
        w_ref[OUTCOME_OFF : OUTCOME_OFF + N_OUTCOMES, :] = outc_ref[:, :]
        w_ref[PAD_COL : PAD_COL + 1, :] = pad_ref[:, :]
        w_ref[PAD_COL + 1 :, :] = jnp.zeros((W_COLS - PAD_COL - 1, D_MODEL), jnp.float32)

    r = lax.broadcasted_iota(jnp.int32, (ROW_BLK, 1), 0) + i * ROW_BLK
    src = r % 64
    dst = (r // 64) % 64
    promo = r % 5
    outc = jnp.clip(r - OUTCOME_TOKEN_BASE, 0, N_OUTCOMES - 1)
    is_pad = r == 0
    is_outcome = r >= OUTCOME_TOKEN_BASE
    is_move = jnp.logical_not(jnp.logical_or(is_pad, is_outcome))

    cols = lax.broadcasted_iota(jnp.int32, (ROW_BLK, W_COLS), 1)
    onehot = (
        jnp.logical_and(
            is_move,
            (cols == src + SRC_OFF)
            | (cols == dst + DST_OFF)
            | (cols == promo + PROMO_OFF),
        )
        | jnp.logical_and(is_outcome, cols == outc + OUTCOME_OFF)
        | jnp.logical_and(is_pad, cols == PAD_COL)
    ).astype(jnp.float32)
    out_ref[:, :] = jnp.dot(onehot, w_ref[:, :], preferred_element_type=jnp.float32)


def _build_table(src_embed, dst_embed, promo_embed, outcome_embed, pad_row):
    full = lambda s: pl.BlockSpec(s, lambda i: tuple(0 for _ in s))
    return pl.pallas_call(
        _build_table_kernel,
        grid=(VOCAB_PAD // ROW_BLK,),
        in_specs=[
            full((64, D_MODEL)),
            full((64, D_MODEL)),
            full((N_OUTCOMES, D_MODEL)),
            full((N_OUTCOMES, D_MODEL)),
            full((1, D_MODEL)),
        ],
        out_specs=pl.BlockSpec((ROW_BLK, D_MODEL), lambda i: (i, 0)),
        out_shape=jax.ShapeDtypeStruct((VOCAB_PAD, D_MODEL), jnp.float32),
        scratch_shapes=[pltpu.VMEM((W_COLS, D_MODEL), jnp.float32)],
    )(src_embed, dst_embed, promo_embed, outcome_embed, pad_row)


def _gather_body(table_hbm, ids_hbm, out_hbm, idx_v, *scratch):
    bufs = scratch[:NBUF]
    gsems = scratch[NBUF : 2 * NBUF]
    osems = scratch[2 * NBUF :]
    wid = lax.axis_index("s") * NUM_CORES + lax.axis_index("c")
    base = wid * IDS_PER_WORKER
    pltpu.sync_copy(ids_hbm.at[wid], idx_v)

    gh = [None] * NUM_CHUNKS
    oh = [None] * NUM_CHUNKS
    for k in range(NUM_CHUNKS):
        b = k % NBUF
        if k >= NBUF:
            oh[k - NBUF].wait()
        gh[k] = pltpu.async_copy(table_hbm.at[idx_v.at[k]], bufs[b], gsems[b])
        if k >= 1:
            pb = (k - 1) % NBUF
            gh[k - 1].wait()
            oh[k - 1] = pltpu.async_copy(
                bufs[pb],
                out_hbm.at[pl.ds(base + (k - 1) * CHUNK, CHUNK)],
                osems[pb],
            )
    last = NUM_CHUNKS - 1
    gh[last].wait()
    oh[last] = pltpu.async_copy(
        bufs[last % NBUF],
        out_hbm.at[pl.ds(base + last * CHUNK, CHUNK)],
        osems[last % NBUF],
    )
    for k in range(max(0, NUM_CHUNKS - NBUF), NUM_CHUNKS):
        oh[k].wait()


_gather_rows = pl.kernel(
    _gather_body,
    mesh=plsc.VectorSubcoreMesh(core_axis_name="c", subcore_axis_name="s"),
    out_type=jax.ShapeDtypeStruct((TOKENS, D_MODEL), jnp.float32),
    scratch_types=(
        [pltpu.VMEM((NUM_CHUNKS, CHUNK), jnp.int32)]
        + [pltpu.VMEM((CHUNK, D_MODEL), jnp.float32) for _ in range(NBUF)]
        + [pltpu.SemaphoreType.DMA for _ in range(2 * NBUF)]
    ),
)


@jax.jit
def kernel(input_ids, src_embed, dst_embed, promo_embed, pad_embed, outcome_embed, decomp_table):
    table = _build_table(
        src_embed, dst_embed, promo_embed, outcome_embed, pad_embed.reshape(1, D_MODEL)
    )
    ids = input_ids.reshape(NUM_WORKERS, NUM_CHUNKS, CHUNK).astype(jnp.int32)
    out = _gather_rows(table, ids)
    return out.reshape(input_ids.shape + (D_MODEL,))


# speedup vs baseline: 1.0165x; 1.0165x over previous
import functools

import jax
import jax.numpy as jnp
from jax import lax
from jax.experimental import pallas as pl
from jax.experimental.pallas import tpu as pltpu
from jax.experimental.pallas import tpu_sc as plsc

D_MODEL = 768
N_OUTCOMES = 5
OUTCOME_TOKEN_BASE = 4273
VOCAB = 4278

ROW_BLK = 1088
VOCAB_PAD = 4352

W_COLS = 144
SRC_OFF = 0
DST_OFF = 64
PROMO_OFF = 128
OUTCOME_OFF = 133
PAD_COL = 138

NUM_CORES = 2
NUM_SUBCORES = 16
NUM_WORKERS = NUM_CORES * NUM_SUBCORES
TOKENS = 4 * 8192
IDS_PER_WORKER = TOKENS // NUM_WORKERS
CHUNK = 80
NBUF = 2
_CHUNKS = [(i * CHUNK, CHUNK) for i in range(12)] + [(12 * CHUNK, 64)]
NUM_CHUNKS = len(_CHUNKS)


def _build_table_kernel(src_ref, dst_ref, promo_ref, outc_ref, pad_ref, out_ref, w_ref):
    i = pl.program_id(0)

    @pl.when(i == 0)
    def _concat_w():
        w_ref[SRC_OFF : SRC_OFF + 64, :] = src_ref[:, :]
        w_ref[DST_OFF : DST_OFF + 64, :] = dst_ref[:, :]
        w_ref[PROMO_OFF : PROMO_OFF + N_OUTCOMES, :] = promo_ref[:, :]
        w_ref[OUTCOME_OFF : OUTCOME_OFF + N_OUTCOMES, :] = outc_ref[:, :]
        w_ref[PAD_COL : PAD_COL + 1, :] = pad_ref[:, :]
        w_ref[PAD_COL + 1 :, :] = jnp.zeros((W_COLS - PAD_COL - 1, D_MODEL), jnp.float32)

    r = lax.broadcasted_iota(jnp.int32, (ROW_BLK, 1), 0) + i * ROW_BLK
    src = r % 64
    dst = (r // 64) % 64
    promo = r % 5
    outc = jnp.clip(r - OUTCOME_TOKEN_BASE, 0, N_OUTCOMES - 1)
    is_pad = r == 0
    is_outcome = r >= OUTCOME_TOKEN_BASE
    is_move = jnp.logical_not(jnp.logical_or(is_pad, is_outcome))

    alt = jnp.where(is_outcome, outc + OUTCOME_OFF, PAD_COL)
    c1 = jnp.where(is_move, src + SRC_OFF, alt)
    c2 = jnp.where(is_move, dst + DST_OFF, alt)
    c3 = jnp.where(is_move, promo + PROMO_OFF, alt)

    cols = lax.broadcasted_iota(jnp.int32, (ROW_BLK, W_COLS), 1)
    onehot = ((cols == c1) | (cols == c2) | (cols == c3)).astype(jnp.float32)
    out_ref[:, :] = jnp.dot(onehot, w_ref[:, :], preferred_element_type=jnp.float32)


def _build_table(src_embed, dst_embed, promo_embed, outcome_embed, pad_row):
    full = lambda s: pl.BlockSpec(s, lambda i: tuple(0 for _ in s))
    return pl.pallas_call(
        _build_table_kernel,
        grid=(VOCAB_PAD // ROW_BLK,),
        in_specs=[
            full((64, D_MODEL)),
            full((64, D_MODEL)),
            full((N_OUTCOMES, D_MODEL)),
            full((N_OUTCOMES, D_MODEL)),
            full((1, D_MODEL)),
        ],
        out_specs=pl.BlockSpec((ROW_BLK, D_MODEL), lambda i: (i, 0)),
        out_shape=jax.ShapeDtypeStruct((VOCAB_PAD, D_MODEL), jnp.float32),
        scratch_shapes=[pltpu.VMEM((W_COLS, D_MODEL), jnp.float32)],
    )(src_embed, dst_embed, promo_embed, outcome_embed, pad_row)


def _gather_body(table_hbm, ids_hbm, out_hbm, idx_v, *scratch):
    bufs = scratch[:NBUF]
    gsems = scratch[NBUF : 2 * NBUF]
    osems = scratch[2 * NBUF :]
    wid = lax.axis_index("s") * NUM_CORES + lax.axis_index("c")
    base = wid * IDS_PER_WORKER
    pltpu.sync_copy(ids_hbm.at[pl.ds(base, IDS_PER_WORKER)], idx_v)

    gh = [None] * NUM_CHUNKS
    oh = [None] * NUM_CHUNKS
    for k in range(NUM_CHUNKS):
        b = k % NBUF
        off, sz = _CHUNKS[k]
        if k >= NBUF:
            oh[k - NBUF].wait()
        gh[k] = pltpu.async_copy(
            table_hbm.at[idx_v.at[pl.ds(off, sz)]],
            bufs[b].at[pl.ds(0, sz)],
            gsems[b],
        )
        if k >= 1:
            pb = (k - 1) % NBUF
            poff, psz = _CHUNKS[k - 1]
            gh[k - 1].wait()
            oh[k - 1] = pltpu.async_copy(
                bufs[pb].at[pl.ds(0, psz)],
                out_hbm.at[pl.ds(base + poff, psz)],
                osems[pb],
            )
    last = NUM_CHUNKS - 1
    loff, lsz = _CHUNKS[last]
    gh[last].wait()
    oh[last] = pltpu.async_copy(
        bufs[last % NBUF].at[pl.ds(0, lsz)],
        out_hbm.at[pl.ds(base + loff, lsz)],
        osems[last % NBUF],
    )
    for k in range(max(0, NUM_CHUNKS - NBUF), NUM_CHUNKS):
        oh[k].wait()


_gather_rows = pl.kernel(
    _gather_body,
    mesh=plsc.VectorSubcoreMesh(core_axis_name="c", subcore_axis_name="s"),
    out_type=jax.ShapeDtypeStruct((TOKENS, D_MODEL), jnp.float32),
    scratch_types=(
        [pltpu.VMEM((IDS_PER_WORKER,), jnp.int32)]
        + [pltpu.VMEM((CHUNK, D_MODEL), jnp.float32) for _ in range(NBUF)]
        + [pltpu.SemaphoreType.DMA for _ in range(2 * NBUF)]
    ),
)


@jax.jit
def kernel(input_ids, src_embed, dst_embed, promo_embed, pad_embed, outcome_embed, decomp_table):
    table = _build_table(
        src_embed, dst_embed, promo_embed, outcome_embed, pad_embed.reshape(1, D_MODEL)
    )
    ids = input_ids.reshape(-1).astype(jnp.int32)
    out = _gather_rows(table, ids)
    return out.reshape(input_ids.shape + (D_MODEL,))


# speedup vs baseline: 1.0216x; 1.0050x over previous
import functools

import jax
import jax.numpy as jnp
from jax import lax
from jax.experimental import pallas as pl
from jax.experimental.pallas import tpu as pltpu
from jax.experimental.pallas import tpu_sc as plsc

D_MODEL = 768
N_OUTCOMES = 5
OUTCOME_TOKEN_BASE = 4273
VOCAB = 4278

ROW_BLK = 1088
VOCAB_PAD = 4352

W_COLS = 144
SRC_OFF = 0
DST_OFF = 64
PROMO_OFF = 128
OUTCOME_OFF = 133
PAD_COL = 138

NUM_CORES = 2
NUM_SUBCORES = 16
NUM_WORKERS = NUM_CORES * NUM_SUBCORES
TOKENS = 4 * 8192
IDS_PER_WORKER = TOKENS // NUM_WORKERS
CHUNK = 48
NBUF = 3
LAG = NBUF - 1
_CHUNKS = [(i * CHUNK, CHUNK) for i in range(21)] + [(21 * CHUNK, 16)]
NUM_CHUNKS = len(_CHUNKS)


def _build_table_kernel(src_ref, dst_ref, promo_ref, outc_ref, pad_ref, out_ref, w_ref):
    i = pl.program_id(0)

    @pl.when(i == 0)
    def _concat_w():
        w_ref[SRC_OFF : SRC_OFF + 64, :] = src_ref[:, :]
        w_ref[DST_OFF : DST_OFF + 64, :] = dst_ref[:, :]
        w_ref[PROMO_OFF : PROMO_OFF + N_OUTCOMES, :] = promo_ref[:, :]
        w_ref[OUTCOME_OFF : OUTCOME_OFF + N_OUTCOMES, :] = outc_ref[:, :]
        w_ref[PAD_COL : PAD_COL + 1, :] = pad_ref[:, :]
        w_ref[PAD_COL + 1 :, :] = jnp.zeros((W_COLS - PAD_COL - 1, D_MODEL), jnp.float32)

    r = lax.broadcasted_iota(jnp.int32, (ROW_BLK, 1), 0) + i * ROW_BLK
    src = r % 64
    dst = (r // 64) % 64
    promo = r % 5
    outc = jnp.clip(r - OUTCOME_TOKEN_BASE, 0, N_OUTCOMES - 1)
    is_pad = r == 0
    is_outcome = r >= OUTCOME_TOKEN_BASE
    is_move = jnp.logical_not(jnp.logical_or(is_pad, is_outcome))

    alt = jnp.where(is_outcome, outc + OUTCOME_OFF, PAD_COL)
    c1 = jnp.where(is_move, src + SRC_OFF, alt)
    c2 = jnp.where(is_move, dst + DST_OFF, alt)
    c3 = jnp.where(is_move, promo + PROMO_OFF, alt)

    cols = lax.broadcasted_iota(jnp.int32, (ROW_BLK, W_COLS), 1)
    onehot = ((cols == c1) | (cols == c2) | (cols == c3)).astype(jnp.float32)
    out_ref[:, :] = jnp.dot(onehot, w_ref[:, :], preferred_element_type=jnp.float32)


def _build_table(src_embed, dst_embed, promo_embed, outcome_embed, pad_row):
    full = lambda s: pl.BlockSpec(s, lambda i: tuple(0 for _ in s))
    return pl.pallas_call(
        _build_table_kernel,
        grid=(VOCAB_PAD // ROW_BLK,),
        in_specs=[
            full((64, D_MODEL)),
            full((64, D_MODEL)),
            full((N_OUTCOMES, D_MODEL)),
            full((N_OUTCOMES, D_MODEL)),
            full((1, D_MODEL)),
        ],
        out_specs=pl.BlockSpec((ROW_BLK, D_MODEL), lambda i: (i, 0)),
        out_shape=jax.ShapeDtypeStruct((VOCAB_PAD, D_MODEL), jnp.float32),
        scratch_shapes=[pltpu.VMEM((W_COLS, D_MODEL), jnp.float32)],
    )(src_embed, dst_embed, promo_embed, outcome_embed, pad_row)


def _gather_body(table_hbm, ids_hbm, out_hbm, idx_v, *scratch):
    bufs = scratch[:NBUF]
    gsems = scratch[NBUF : 2 * NBUF]
    osems = scratch[2 * NBUF :]
    wid = lax.axis_index("s") * NUM_CORES + lax.axis_index("c")
    base = wid * IDS_PER_WORKER
    pltpu.sync_copy(ids_hbm.at[pl.ds(base, IDS_PER_WORKER)], idx_v)

    def _start_out(j):
        joff, jsz = _CHUNKS[j]
        jb = j % NBUF
        gh[j].wait()
        oh[j] = pltpu.async_copy(
            bufs[jb].at[pl.ds(0, jsz)],
            out_hbm.at[pl.ds(base + joff, jsz)],
            osems[jb],
        )

    gh = [None] * NUM_CHUNKS
    oh = [None] * NUM_CHUNKS
    for k in range(NUM_CHUNKS):
        b = k % NBUF
        off, sz = _CHUNKS[k]
        if k >= NBUF:
            oh[k - NBUF].wait()
        gh[k] = pltpu.async_copy(
            table_hbm.at[idx_v.at[pl.ds(off, sz)]],
            bufs[b].at[pl.ds(0, sz)],
            gsems[b],
        )
        if k >= LAG:
            _start_out(k - LAG)
    for j in range(NUM_CHUNKS - LAG, NUM_CHUNKS):
        _start_out(j)
    for j in range(max(0, NUM_CHUNKS - NBUF), NUM_CHUNKS):
        oh[j].wait()


_gather_rows = pl.kernel(
    _gather_body,
    mesh=plsc.VectorSubcoreMesh(core_axis_name="c", subcore_axis_name="s"),
    out_type=jax.ShapeDtypeStruct((TOKENS, D_MODEL), jnp.float32),
    scratch_types=(
        [pltpu.VMEM((IDS_PER_WORKER,), jnp.int32)]
        + [pltpu.VMEM((CHUNK, D_MODEL), jnp.float32) for _ in range(NBUF)]
        + [pltpu.SemaphoreType.DMA for _ in range(2 * NBUF)]
    ),
)


@jax.jit
def kernel(input_ids, src_embed, dst_embed, promo_embed, pad_embed, outcome_embed, decomp_table):
    table = _build_table(
        src_embed, dst_embed, promo_embed, outcome_embed, pad_embed.reshape(1, D_MODEL)
    )
    ids = input_ids.reshape(-1).astype(jnp.int32)
    out = _gather_rows(table, ids)
    return out.reshape(input_ids.shape + (D_MODEL,))
